# Initial kernel scaffold; baseline (speedup 1.0000x reference)
import jax
import jax.numpy as jnp
from jax.experimental import pallas as pl


def kernel(x, edge_index, batch, W1, b1, g1, be1, W2, b2, g2, be2, gg, bg, Wr, br, Wc, bc):
    raise NotImplementedError("write your pallas kernel here")


# speedup vs baseline: 4.0871x; 4.0871x over previous
import functools

import jax
import jax.numpy as jnp
from jax import lax
from jax.experimental import pallas as pl
from jax.experimental.pallas import tpu as pltpu
from jax.experimental.pallas import tpu_sc as plsc

N = 10000
E = 320000
D = 128
G = 10
EPS = 1e-5

NCORE = 2
NSUB = 16
NTILE = NCORE * NSUB
ET = E // NTILE
CH = 80
NIT = ET // CH
RPT = 632
RPT_LAST = N - (NSUB - 1) * RPT
ZR = 8
NZIT = RPT // ZR
NZIT_LAST = RPT_LAST // ZR

R = 400
NB = N // R

_f32 = jnp.float32


def _mesh():
    return plsc.VectorSubcoreMesh(core_axis_name="c", subcore_axis_name="s",
                                  num_cores=NCORE, num_subcores=NSUB)


def _sc_degrees(src, dst):
    @functools.partial(
        pl.kernel,
        out_type=(jax.ShapeDtypeStruct((NCORE * N, D), _f32),
                  jax.ShapeDtypeStruct((NCORE * N, D), _f32)),
        mesh=_mesh(),
        scratch_types=[
            pltpu.VMEM((CH, D), _f32),
            pltpu.VMEM((ZR, D), _f32),
            pltpu.VMEM((CH,), jnp.int32),
            pltpu.VMEM_SHARED((N, D), _f32),
        ],
    )
    def deg_kernel(src_hbm, dst_hbm, degs_out, degd_out,
                   ones_v, zb, idxv, acc):
        c = lax.axis_index("c")
        s = lax.axis_index("s")
        g = c * NSUB + s
        zero16 = jnp.zeros((16,), _f32)
        one16 = jnp.ones((16,), _f32)
        for r in range(ZR):
            for j in range(D // 16):
                zb[r, pl.ds(j * 16, 16)] = zero16
        for r in range(CH):
            for j in range(D // 16):
                ones_v[r, pl.ds(j * 16, 16)] = one16
        row0 = s * RPT
        nzit = jnp.where(s == NSUB - 1, NZIT_LAST, NZIT)
        base = g * ET

        def zero_acc():
            def zbody(i, carry):
                pltpu.sync_copy(zb, acc.at[pl.ds(row0 + i * ZR, ZR)])
                return carry
            lax.fori_loop(0, nzit, zbody, None)

        def hist(edge_hbm):
            def ebody(i, carry):
                off = base + i * CH
                pltpu.sync_copy(edge_hbm.at[pl.ds(off, CH)], idxv)
                pltpu.sync_copy(ones_v, acc.at[idxv], add=True)
                return carry
            lax.fori_loop(0, NIT, ebody, None)

        def copy_out(out_hbm):
            @pl.when(s < NSUB - 1)
            def _():
                pltpu.sync_copy(acc.at[pl.ds(row0, RPT)],
                                out_hbm.at[pl.ds(c * N + row0, RPT)])

            @pl.when(s == NSUB - 1)
            def _():
                pltpu.sync_copy(acc.at[pl.ds(row0, RPT_LAST)],
                                out_hbm.at[pl.ds(c * N + row0, RPT_LAST)])

        zero_acc()
        plsc.subcore_barrier()
        hist(src_hbm)
        plsc.subcore_barrier()
        copy_out(degs_out)
        zero_acc()
        plsc.subcore_barrier()
        hist(dst_hbm)
        plsc.subcore_barrier()
        copy_out(degd_out)

    return deg_kernel(src, dst)


def _sc_spmm(hmat, src, dst):
    @functools.partial(
        pl.kernel,
        out_type=jax.ShapeDtypeStruct((NCORE * N, D), _f32),
        mesh=_mesh(),
        scratch_types=[
            pltpu.VMEM((ZR, D), _f32),
            pltpu.VMEM((CH,), jnp.int32),
            pltpu.VMEM((CH,), jnp.int32),
            pltpu.VMEM((CH, D), _f32),
            pltpu.VMEM_SHARED((N, D), _f32),
            pltpu.SemaphoreType.DMA,
        ],
    )
    def spmm_kernel(h_hbm, src_hbm, dst_hbm, out_hbm,
                    zb, sidx, didx, rows, acc, sem):
        c = lax.axis_index("c")
        s = lax.axis_index("s")
        g = c * NSUB + s
        zero16 = jnp.zeros((16,), _f32)
        for r in range(ZR):
            for j in range(D // 16):
                zb[r, pl.ds(j * 16, 16)] = zero16
        row0 = s * RPT
        nzit = jnp.where(s == NSUB - 1, NZIT_LAST, NZIT)

        def zbody(i, carry):
            pltpu.sync_copy(zb, acc.at[pl.ds(row0 + i * ZR, ZR)])
            return carry
        lax.fori_loop(0, nzit, zbody, None)
        plsc.subcore_barrier()

        base = g * ET

        def ebody(i, carry):
            off = base + i * CH
            pltpu.sync_copy(src_hbm.at[pl.ds(off, CH)], sidx)
            pltpu.sync_copy(dst_hbm.at[pl.ds(off, CH)], didx)
            pltpu.async_copy(h_hbm.at[sidx], rows, sem).wait()
            pltpu.sync_copy(rows, acc.at[didx], add=True)
            return carry
        lax.fori_loop(0, NIT, ebody, None)
        plsc.subcore_barrier()

        @pl.when(s < NSUB - 1)
        def _():
            pltpu.sync_copy(acc.at[pl.ds(row0, RPT)],
                            out_hbm.at[pl.ds(c * N + row0, RPT)])

        @pl.when(s == NSUB - 1)
        def _():
            pltpu.sync_copy(acc.at[pl.ds(row0, RPT_LAST)],
                            out_hbm.at[pl.ds(c * N + row0, RPT_LAST)])

    return spmm_kernel(hmat, src, dst)


def _dot(a, b):
    return lax.dot_general(a, b, (((1,), (0,)), ((), ())),
                           preferred_element_type=_f32)


def _tc_prep(x, degs, degd):
    def body(x_ref, ds_ref, dd_ref, xs_ref, ii_ref, io_ref):
        dsum_o = ds_ref[0] + ds_ref[1]
        inv_o = lax.rsqrt(jnp.maximum(dsum_o, 1.0))
        dsum_i = dd_ref[0] + dd_ref[1]
        inv_i = lax.rsqrt(jnp.maximum(dsum_i, 1.0))
        xs_ref[...] = x_ref[...] * inv_o
        ii_ref[...] = inv_i
        io_ref[...] = inv_o

    return pl.pallas_call(
        body,
        grid=(NB,),
        in_specs=[
            pl.BlockSpec((R, D), lambda j: (j, 0)),
            pl.BlockSpec((NCORE, R, D), lambda j: (0, j, 0)),
            pl.BlockSpec((NCORE, R, D), lambda j: (0, j, 0)),
        ],
        out_specs=[pl.BlockSpec((R, D), lambda j: (j, 0))] * 3,
        out_shape=[jax.ShapeDtypeStruct((N, D), _f32)] * 3,
    )(x, degs, degd)


def _tc_post1(aggp, inv_in, inv_out, W, b, gamma, beta):
    def body(agg_ref, ii_ref, io_ref, w_ref, b_ref, g_ref, be_ref,
             out_ref, acc_ref):
        ph = pl.program_id(0)
        j = pl.program_id(1)
        a = (agg_ref[0] + agg_ref[1]) * ii_ref[...]
        p = _dot(a, w_ref[...]) + b_ref[...]

        @pl.when((ph == 0) & (j == 0))
        def _():
            acc_ref[...] = jnp.zeros_like(acc_ref)

        @pl.when(ph == 0)
        def _():
            acc_ref[0:1] = acc_ref[0:1] + jnp.sum(p, axis=0, keepdims=True)
            acc_ref[1:2] = acc_ref[1:2] + jnp.sum(p * p, axis=0, keepdims=True)
            out_ref[...] = p

        @pl.when(ph == 1)
        def _():
            mu = acc_ref[0:1] / N
            var = acc_ref[1:2] / N - mu * mu
            rstd = lax.rsqrt(var + EPS)
            h = jnp.maximum((p - mu) * rstd * g_ref[...] + be_ref[...], 0.0)
            out_ref[...] = h * io_ref[...]

    return pl.pallas_call(
        body,
        grid=(2, NB),
        in_specs=[
            pl.BlockSpec((NCORE, R, D), lambda p, j: (0, j, 0)),
            pl.BlockSpec((R, D), lambda p, j: (j, 0)),
            pl.BlockSpec((R, D), lambda p, j: (j, 0)),
            pl.BlockSpec((D, D), lambda p, j: (0, 0)),
            pl.BlockSpec((1, D), lambda p, j: (0, 0)),
            pl.BlockSpec((1, D), lambda p, j: (0, 0)),
            pl.BlockSpec((1, D), lambda p, j: (0, 0)),
        ],
        out_specs=pl.BlockSpec((R, D), lambda p, j: (j, 0)),
        out_shape=jax.ShapeDtypeStruct((N, D), _f32),
        scratch_shapes=[pltpu.VMEM((8, D), _f32)],
    )(aggp, inv_in, inv_out, W, b, gamma, beta)


def _tc_post2(aggp, inv_in, batch_r, W, b, gamma, beta, ggv, bgv,
              wr_p, br_p, wc_p, bc_p):
    def body(agg_ref, ii_ref, bt_ref, w_ref, b_ref, g_ref, be_ref,
             gg_ref, bg_ref, wr_ref, br_ref, wc_ref, bc_ref,
             h_ref, y_ref, cc_ref, acc_ref, gsum_ref, gcnt_ref):
        ph = pl.program_id(0)
        j = pl.program_id(1)
        a = (agg_ref[0] + agg_ref[1]) * ii_ref[...]
        p = _dot(a, w_ref[...]) + b_ref[...]

        @pl.when((ph == 0) & (j == 0))
        def _():
            acc_ref[...] = jnp.zeros_like(acc_ref)
            gsum_ref[...] = jnp.zeros_like(gsum_ref)
            gcnt_ref[...] = jnp.zeros_like(gcnt_ref)

        @pl.when(ph == 0)
        def _():
            acc_ref[0:1] = acc_ref[0:1] + jnp.sum(p, axis=0, keepdims=True)
            acc_ref[1:2] = acc_ref[1:2] + jnp.sum(p * p, axis=0, keepdims=True)
            h_ref[...] = p

        @pl.when(ph == 1)
        def _():
            mu = acc_ref[0:1] / N
            var = acc_ref[1:2] / N - mu * mu
            rstd = lax.rsqrt(var + EPS)
            h = jnp.maximum((p - mu) * rstd * g_ref[...] + be_ref[...], 0.0)
            h_ref[...] = h
            bt = bt_ref[0]
            gi = lax.broadcasted_iota(jnp.int32, (16, R), 0)
            oh = (gi == jnp.broadcast_to(bt, (16, R))).astype(_f32)
            gsum_ref[...] = gsum_ref[...] + _dot(oh, h)
            gcnt_ref[...] = gcnt_ref[...] + jnp.broadcast_to(
                jnp.sum(oh, axis=1, keepdims=True), (16, D))

        @pl.when((ph == 1) & (j == NB - 1))
        def _():
            cnt = jnp.maximum(gcnt_ref[...], 1.0)
            gemb = gsum_ref[...] / cnt
            rmask = (lax.broadcasted_iota(jnp.int32, (16, D), 0) < G)
            rmaskf = rmask.astype(_f32)
            gm = jnp.sum(gemb * rmaskf, axis=0, keepdims=True) / G
            gv = jnp.sum(((gemb - gm) ** 2) * rmaskf, axis=0,
                         keepdims=True) / G
            gn = (gemb - gm) * lax.rsqrt(gv + EPS) * gg_ref[...] + bg_ref[...]
            y_ref[...] = _dot(gn, wr_ref[...]) + br_ref[...]
            cc_ref[...] = _dot(gn, wc_ref[...]) + bc_ref[...]

    return pl.pallas_call(
        body,
        grid=(2, NB),
        in_specs=[
            pl.BlockSpec((NCORE, R, D), lambda p, j: (0, j, 0)),
            pl.BlockSpec((R, D), lambda p, j: (j, 0)),
            pl.BlockSpec((1, 1, R), lambda p, j: (j, 0, 0)),
            pl.BlockSpec((D, D), lambda p, j: (0, 0)),
            pl.BlockSpec((1, D), lambda p, j: (0, 0)),
            pl.BlockSpec((1, D), lambda p, j: (0, 0)),
            pl.BlockSpec((1, D), lambda p, j: (0, 0)),
            pl.BlockSpec((1, D), lambda p, j: (0, 0)),
            pl.BlockSpec((1, D), lambda p, j: (0, 0)),
            pl.BlockSpec((D, D), lambda p, j: (0, 0)),
            pl.BlockSpec((1, D), lambda p, j: (0, 0)),
            pl.BlockSpec((D, D), lambda p, j: (0, 0)),
            pl.BlockSpec((1, D), lambda p, j: (0, 0)),
        ],
        out_specs=[
            pl.BlockSpec((R, D), lambda p, j: (j, 0)),
            pl.BlockSpec((16, D), lambda p, j: (0, 0)),
            pl.BlockSpec((16, D), lambda p, j: (0, 0)),
        ],
        out_shape=[
            jax.ShapeDtypeStruct((N, D), _f32),
            jax.ShapeDtypeStruct((16, D), _f32),
            jax.ShapeDtypeStruct((16, D), _f32),
        ],
        scratch_shapes=[pltpu.VMEM((8, D), _f32),
                        pltpu.VMEM((16, D), _f32),
                        pltpu.VMEM((16, D), _f32)],
    )(aggp, inv_in, batch_r, W, b, gamma, beta, ggv, bgv,
      wr_p, br_p, wc_p, bc_p)


def kernel(x, edge_index, batch, W1, b1, g1, be1, W2, b2, g2, be2,
           gg, bg, Wr, br, Wc, bc):
    src = edge_index[0]
    dst = edge_index[1]

    degs2, degd2 = _sc_degrees(src, dst)
    degs = degs2.reshape(NCORE, N, D)
    degd = degd2.reshape(NCORE, N, D)

    xs, inv_in, inv_out = _tc_prep(x, degs, degd)

    agg1 = _sc_spmm(xs, src, dst).reshape(NCORE, N, D)
    h1s = _tc_post1(agg1, inv_in, inv_out, W1,
                    b1.reshape(1, D), g1.reshape(1, D), be1.reshape(1, D))

    agg2 = _sc_spmm(h1s, src, dst).reshape(NCORE, N, D)

    nout = Wr.shape[1]
    ncpt = Wc.shape[1]
    wr_p = jnp.pad(Wr, ((0, 0), (0, D - nout)))
    br_p = jnp.pad(br, (0, D - nout)).reshape(1, D)
    wc_p = jnp.pad(Wc, ((0, 0), (0, D - ncpt)))
    bc_p = jnp.pad(bc, (0, D - ncpt)).reshape(1, D)
    batch_r = batch.reshape(NB, 1, R)

    h, y_f, c_f = _tc_post2(agg2, inv_in, batch_r, W2,
                            b2.reshape(1, D), g2.reshape(1, D),
                            be2.reshape(1, D), gg.reshape(1, D),
                            bg.reshape(1, D), wr_p, br_p, wc_p, bc_p)
    y = y_f[:G, :nout]
    concept = c_f[:G, :ncpt]
    return (h, y, concept)
